# baseline (speedup 1.0000x reference)
import functools

import jax
import jax.numpy as jnp
from jax import lax
from jax.experimental import pallas as pl
from jax.experimental.pallas import tpu as pltpu
from jax.experimental.pallas import tpu_sc as plsc

B = 16384
D = 128
L = 16
NSUB = D // L

NC = 2
NS = 16
NW = NC * NS
BPW = B // NW

CH = 128
NCH = BPW // CH


def _transh_body(idx_hbm, ent_hbm, relcat_hbm, out_hbm,
                 idxv, hbuf, tbuf, wrbuf, obuf, gsem):
    cid = lax.axis_index("c")
    sid = lax.axis_index("s")
    wid = sid * NC + cid

    pltpu.sync_copy(idx_hbm.at[wid], idxv)

    base = wid * BPW
    for c in range(NCH):
        hsl = pl.ds(c * CH, CH)
        tsl = pl.ds(BPW + c * CH, CH)
        rsl = pl.ds(2 * BPW + c * CH, CH)
        cps = (
            pltpu.async_copy(ent_hbm.at[idxv.at[hsl]], hbuf, gsem),
            pltpu.async_copy(ent_hbm.at[idxv.at[tsl]], tbuf, gsem),
            pltpu.async_copy(relcat_hbm.at[idxv.at[rsl]], wrbuf, gsem),
        )
        for cp in cps:
            cp.wait()

        def row(i, carry):
            acc = jnp.zeros((L,), jnp.float32)
            hmts = []
            ws = []
            for j in range(NSUB):
                csl = pl.ds(j * L, L)
                h = hbuf[i, csl]
                t = tbuf[i, csl]
                w = wrbuf[i, csl]
                hmt = h - t
                acc = acc + hmt * w
                hmts.append(hmt)
                ws.append(w)
            d = jnp.sum(acc)
            for j in range(NSUB):
                csl = pl.ds(j * L, L)
                r = wrbuf[i, pl.ds(D + j * L, L)]
                obuf[i, csl] = hmts[j] + r - ws[j] * d
            return carry

        lax.fori_loop(0, CH, row, 0)
        pltpu.sync_copy(obuf, out_hbm.at[pl.ds(base + c * CH, CH)])


_transh = functools.partial(
    pl.kernel,
    out_type=jax.ShapeDtypeStruct((B, D), jnp.float32),
    mesh=plsc.VectorSubcoreMesh(core_axis_name="c", subcore_axis_name="s"),
    compiler_params=pltpu.CompilerParams(needs_layout_passes=False),
    scratch_types=[
        pltpu.VMEM((3 * BPW,), jnp.int32),
        pltpu.VMEM((CH, D), jnp.float32),
        pltpu.VMEM((CH, D), jnp.float32),
        pltpu.VMEM((CH, 2 * D), jnp.float32),
        pltpu.VMEM((CH, D), jnp.float32),
        pltpu.SemaphoreType.DMA,
    ],
)(_transh_body)


def kernel(head, relation, tail, ent_emb, rel_emb, rel_hyper):
    rel_cat = jnp.concatenate([rel_hyper, rel_emb], axis=1)
    idx = jnp.concatenate(
        [head.reshape(NW, BPW), tail.reshape(NW, BPW),
         relation.reshape(NW, BPW)], axis=1)
    return _transh(idx, ent_emb, rel_cat)


# speedup vs baseline: 1.5728x; 1.5728x over previous
import functools

import jax
import jax.numpy as jnp
from jax import lax
from jax.experimental import pallas as pl
from jax.experimental.pallas import tpu as pltpu
from jax.experimental.pallas import tpu_sc as plsc

B = 16384
D = 128
L = 16
NSUB = D // L

NC = 2
NS = 16
NW = NC * NS
BPW = B // NW

CH = 128
NCH = BPW // CH


def _transh_body(head_hbm, rel_hbm, tail_hbm, ent_hbm, rele_hbm, relh_hbm,
                 out_hbm, hidx, tidx, ridx, hbuf, tbuf, wbuf, rbuf, obuf, sem):
    cid = lax.axis_index("c")
    sid = lax.axis_index("s")
    wid = sid * NC + cid
    base = wid * BPW

    pltpu.sync_copy(head_hbm.at[pl.ds(base, BPW)], hidx)
    pltpu.sync_copy(tail_hbm.at[pl.ds(base, BPW)], tidx)
    pltpu.sync_copy(rel_hbm.at[pl.ds(base, BPW)], ridx)

    for c in range(NCH):
        isl = pl.ds(c * CH, CH)
        cps = (
            pltpu.async_copy(ent_hbm.at[hidx.at[isl]], hbuf, sem),
            pltpu.async_copy(ent_hbm.at[tidx.at[isl]], tbuf, sem),
            pltpu.async_copy(relh_hbm.at[ridx.at[isl]], wbuf, sem),
            pltpu.async_copy(rele_hbm.at[ridx.at[isl]], rbuf, sem),
        )
        for cp in cps:
            cp.wait()

        def row(i, carry):
            acc = jnp.zeros((L,), jnp.float32)
            hmts = []
            ws = []
            for j in range(NSUB):
                csl = pl.ds(j * L, L)
                h = hbuf[i, csl]
                t = tbuf[i, csl]
                w = wbuf[i, csl]
                hmt = h - t
                acc = acc + hmt * w
                hmts.append(hmt)
                ws.append(w)
            d = jnp.sum(acc)
            for j in range(NSUB):
                csl = pl.ds(j * L, L)
                r = rbuf[i, csl]
                obuf[i, csl] = hmts[j] + r - ws[j] * d
            return carry

        lax.fori_loop(0, CH, row, 0)
        pltpu.sync_copy(obuf, out_hbm.at[pl.ds(base + c * CH, CH)])


_transh = functools.partial(
    pl.kernel,
    out_type=jax.ShapeDtypeStruct((B, D), jnp.float32),
    mesh=plsc.VectorSubcoreMesh(core_axis_name="c", subcore_axis_name="s"),
    compiler_params=pltpu.CompilerParams(needs_layout_passes=False),
    scratch_types=[
        pltpu.VMEM((BPW,), jnp.int32),
        pltpu.VMEM((BPW,), jnp.int32),
        pltpu.VMEM((BPW,), jnp.int32),
        pltpu.VMEM((CH, D), jnp.float32),
        pltpu.VMEM((CH, D), jnp.float32),
        pltpu.VMEM((CH, D), jnp.float32),
        pltpu.VMEM((CH, D), jnp.float32),
        pltpu.VMEM((CH, D), jnp.float32),
        pltpu.SemaphoreType.DMA,
    ],
)(_transh_body)


def kernel(head, relation, tail, ent_emb, rel_emb, rel_hyper):
    return _transh(head, relation, tail, ent_emb, rel_emb, rel_hyper)


# speedup vs baseline: 1.6094x; 1.0232x over previous
import functools

import jax
import jax.numpy as jnp
from jax import lax
from jax.experimental import pallas as pl
from jax.experimental.pallas import tpu as pltpu
from jax.experimental.pallas import tpu_sc as plsc

B = 16384
D = 128
L = 16
NSUB = D // L

NC = 2
NS = 16
NW = NC * NS
BPW = B // NW

CH = 128
NCH = BPW // CH


def _transh_body(head_hbm, rel_hbm, tail_hbm, ent_hbm, rele_hbm, relh_hbm,
                 out_hbm, hidx, tidx, ridx, hbuf, tbuf, wbuf, rbuf, obuf, sem):
    cid = lax.axis_index("c")
    sid = lax.axis_index("s")
    wid = sid * NC + cid
    base = wid * BPW

    pltpu.sync_copy(head_hbm.at[pl.ds(base, BPW)], hidx)
    pltpu.sync_copy(tail_hbm.at[pl.ds(base, BPW)], tidx)
    pltpu.sync_copy(rel_hbm.at[pl.ds(base, BPW)], ridx)

    def chunk(c, carry):
        isl = pl.ds(c * CH, CH)
        cps = (
            pltpu.async_copy(ent_hbm.at[hidx.at[isl]], hbuf, sem),
            pltpu.async_copy(ent_hbm.at[tidx.at[isl]], tbuf, sem),
            pltpu.async_copy(relh_hbm.at[ridx.at[isl]], wbuf, sem),
            pltpu.async_copy(rele_hbm.at[ridx.at[isl]], rbuf, sem),
        )
        for cp in cps:
            cp.wait()

        def row(i, rcarry):
            acc = jnp.zeros((L,), jnp.float32)
            hmts = []
            ws = []
            for j in range(NSUB):
                csl = pl.ds(j * L, L)
                h = hbuf[i, csl]
                t = tbuf[i, csl]
                w = wbuf[i, csl]
                hmt = h - t
                acc = acc + hmt * w
                hmts.append(hmt)
                ws.append(w)
            d = jnp.sum(acc)
            for j in range(NSUB):
                csl = pl.ds(j * L, L)
                r = rbuf[i, csl]
                obuf[i, csl] = hmts[j] + r - ws[j] * d
            return rcarry

        lax.fori_loop(0, CH, row, 0)
        pltpu.sync_copy(obuf, out_hbm.at[pl.ds(base + c * CH, CH)])
        return carry

    lax.fori_loop(0, NCH, chunk, 0)


_transh = functools.partial(
    pl.kernel,
    out_type=jax.ShapeDtypeStruct((B, D), jnp.float32),
    mesh=plsc.VectorSubcoreMesh(core_axis_name="c", subcore_axis_name="s"),
    compiler_params=pltpu.CompilerParams(needs_layout_passes=False),
    scratch_types=[
        pltpu.VMEM((BPW,), jnp.int32),
        pltpu.VMEM((BPW,), jnp.int32),
        pltpu.VMEM((BPW,), jnp.int32),
        pltpu.VMEM((CH, D), jnp.float32),
        pltpu.VMEM((CH, D), jnp.float32),
        pltpu.VMEM((CH, D), jnp.float32),
        pltpu.VMEM((CH, D), jnp.float32),
        pltpu.VMEM((CH, D), jnp.float32),
        pltpu.SemaphoreType.DMA,
    ],
)(_transh_body)


def kernel(head, relation, tail, ent_emb, rel_emb, rel_hyper):
    return _transh(head, relation, tail, ent_emb, rel_emb, rel_hyper)
